# baseline (speedup 1.0000x reference)
import functools

import jax
import jax.numpy as jnp
from jax import lax
from jax.experimental import pallas as pl
from jax.experimental.pallas import tpu as pltpu
from jax.experimental.pallas import tpu_sc as plsc

_D = 64
_CHUNK = 128


@functools.lru_cache(maxsize=None)
def _build(B: int):
    info = plsc.get_sparse_core_info()
    nc, ns = info.num_cores, info.num_subcores
    nw = nc * ns
    b_per_w = B // nw
    n_chunks = b_per_w // _CHUNK
    mesh = plsc.VectorSubcoreMesh(core_axis_name="c", subcore_axis_name="s")

    @functools.partial(
        pl.kernel,
        mesh=mesh,
        compiler_params=pltpu.CompilerParams(use_tc_tiling_on_sc=False),
        out_type=jax.ShapeDtypeStruct((B, _D), jnp.float32),
        scratch_types=[
            pltpu.VMEM((b_per_w,), jnp.int32),
            pltpu.VMEM((_CHUNK, _D), jnp.float32),
            pltpu.VMEM((16,), jnp.float32),
            pltpu.SemaphoreType.DMA,
        ],
    )
    def k(idx_hbm, table_hbm, scale_hbm, out_hbm, idx_v, rows_v, scale_v, sem):
        wid = lax.axis_index("s") * nc + lax.axis_index("c")
        base = wid * b_per_w
        pltpu.sync_copy(idx_hbm.at[pl.ds(base, b_per_w)], idx_v)
        pltpu.sync_copy(scale_hbm, scale_v)
        sv = scale_v[...]

        def chunk_body(j, carry):
            pltpu.async_copy(
                table_hbm.at[idx_v.at[pl.ds(j * _CHUNK, _CHUNK)]],
                rows_v, sem).wait()

            def row_body(r, c2):
                for c in range(_D // 16):
                    rows_v[r, pl.ds(c * 16, 16)] = (
                        rows_v[r, pl.ds(c * 16, 16)] * sv)
                return c2

            lax.fori_loop(0, _CHUNK, row_body, 0)
            pltpu.sync_copy(rows_v, out_hbm.at[pl.ds(base + j * _CHUNK, _CHUNK)])
            return carry

        lax.fori_loop(0, n_chunks, chunk_body, 0)

    return k


def kernel(token_ids, embed_weight, scale):
    shape = token_ids.shape
    idx = token_ids.reshape(-1).astype(jnp.int32)
    scale_vec = jnp.broadcast_to(scale.astype(jnp.float32), (16,))
    out = _build(idx.shape[0])(idx, embed_weight, scale_vec)
    return out.reshape(*shape, _D)


# speedup vs baseline: 1.0702x; 1.0702x over previous
import functools

import jax
import jax.numpy as jnp
from jax import lax
from jax.experimental import pallas as pl
from jax.experimental.pallas import tpu as pltpu
from jax.experimental.pallas import tpu_sc as plsc

_D = 64
_CHUNK = 128
_NBUF = 4


@functools.lru_cache(maxsize=None)
def _build(B: int):
    info = plsc.get_sparse_core_info()
    nc, ns = info.num_cores, info.num_subcores
    nw = nc * ns
    b_per_w = B // nw
    n_chunks = b_per_w // _CHUNK
    mesh = plsc.VectorSubcoreMesh(core_axis_name="c", subcore_axis_name="s")

    @functools.partial(
        pl.kernel,
        mesh=mesh,
        compiler_params=pltpu.CompilerParams(use_tc_tiling_on_sc=False),
        out_type=jax.ShapeDtypeStruct((B, _D), jnp.float32),
        scratch_types=[
            pltpu.VMEM((b_per_w,), jnp.int32),
            pltpu.VMEM((_NBUF, _CHUNK, _D), jnp.float32),
            pltpu.VMEM((16,), jnp.float32),
            pltpu.SemaphoreType.DMA((_NBUF,)),
            pltpu.SemaphoreType.DMA((_NBUF,)),
        ],
    )
    def k(idx_hbm, table_hbm, scale_hbm, out_hbm, idx_v, rows_v, scale_v,
          in_sem, out_sem):
        wid = lax.axis_index("s") * nc + lax.axis_index("c")
        base = wid * b_per_w
        pltpu.sync_copy(idx_hbm.at[pl.ds(base, b_per_w)], idx_v)
        pltpu.sync_copy(scale_hbm, scale_v)
        sv = scale_v[...]

        def gather(j, b):
            return pltpu.make_async_copy(
                table_hbm.at[idx_v.at[pl.ds(j * _CHUNK, _CHUNK)]],
                rows_v.at[b], in_sem.at[b])

        out2d = out_hbm

        def writeout(j, b):
            return pltpu.make_async_copy(
                rows_v.at[b],
                out2d.at[pl.ds(base + j * _CHUNK, _CHUNK)],
                out_sem.at[b])

        for b in range(_NBUF):
            gather(b, b).start()

        for j in range(n_chunks):
            b = j % _NBUF
            gather(j, b).wait()

            def row_body(r, c2):
                for rr in range(2):
                    for c in range(_D // 16):
                        rows_v[b, 2 * r + rr, pl.ds(c * 16, 16)] = (
                            rows_v[b, 2 * r + rr, pl.ds(c * 16, 16)] * sv)
                return c2

            lax.fori_loop(0, _CHUNK // 2, row_body, 0, unroll=2)
            writeout(j, b).start()
            nj = j + _NBUF
            if nj < n_chunks:
                writeout(j, b).wait()
                gather(nj, b).start()

        for j in range(n_chunks - _NBUF, n_chunks):
            writeout(j, j % _NBUF).wait()

    return k


def kernel(token_ids, embed_weight, scale):
    shape = token_ids.shape
    idx = token_ids.reshape(-1).astype(jnp.int32)
    scale_vec = jnp.broadcast_to(scale.astype(jnp.float32), (16,))
    out = _build(idx.shape[0])(idx, embed_weight, scale_vec)
    return out.reshape(*shape, _D)


# speedup vs baseline: 1.4766x; 1.3797x over previous
import functools

import jax
import jax.numpy as jnp
from jax import lax
from jax.experimental import pallas as pl
from jax.experimental.pallas import tpu as pltpu
from jax.experimental.pallas import tpu_sc as plsc

_D = 64
_CHUNK = 128
_NBUF = 2
_INFLIGHT = 16


@functools.lru_cache(maxsize=None)
def _build(B: int):
    info = plsc.get_sparse_core_info()
    nc, ns = info.num_cores, info.num_subcores
    nw = nc * ns
    b_per_w = B // nw
    n_chunks = b_per_w // _CHUNK
    mesh = plsc.VectorSubcoreMesh(core_axis_name="c", subcore_axis_name="s")

    @functools.partial(
        pl.kernel,
        mesh=mesh,
        compiler_params=pltpu.CompilerParams(use_tc_tiling_on_sc=True),
        out_type=jax.ShapeDtypeStruct((B, _D), jnp.float32),
        scratch_types=[
            pltpu.VMEM((b_per_w,), jnp.int32),
            pltpu.VMEM((_NBUF, _CHUNK, _D), jnp.float32),
            pltpu.VMEM((16,), jnp.float32),
            pltpu.SemaphoreType.DMA((_NBUF,)),
            pltpu.SemaphoreType.DMA((_NBUF,)),
        ],
    )
    def k(idx_hbm, table_hbm, scale_hbm, out_hbm, idx_v, rows_v,
          scale_v, in_sem, out_sem):
        wid = lax.axis_index("s") * nc + lax.axis_index("c")
        base = wid * b_per_w
        pltpu.sync_copy(idx_hbm.at[pl.ds(base, b_per_w)], idx_v)
        pltpu.sync_copy(scale_hbm, scale_v)
        sv = scale_v[...]

        def gather_chunk(j, b):
            def fire(i, c2):
                iv = idx_v[pl.ds(j * _CHUNK + i * 16, 16)]
                for q in range(16):
                    row = iv[q]
                    pltpu.make_async_copy(
                        table_hbm.at[pl.ds(row, 1)],
                        rows_v.at[b, pl.ds(i * 16 + q, 1)],
                        in_sem.at[b]).start()
                return c2
            lax.fori_loop(0, _CHUNK // 16, fire, 0)

        def wait_chunk(b):
            pltpu.make_async_copy(
                table_hbm.at[pl.ds(0, _CHUNK)],
                rows_v.at[b], in_sem.at[b]).wait()

        def writeout(j, b):
            return pltpu.make_async_copy(
                rows_v.at[b],
                out_hbm.at[pl.ds(base + j * _CHUNK, _CHUNK)],
                out_sem.at[b])

        for b in range(_NBUF):
            gather_chunk(b, b)

        for j in range(n_chunks):
            b = j % _NBUF
            wait_chunk(b)

            def row_body(r, c2):
                for rr in range(2):
                    for c in range(_D // 16):
                        rows_v[b, 2 * r + rr, pl.ds(c * 16, 16)] = (
                            rows_v[b, 2 * r + rr, pl.ds(c * 16, 16)] * sv)
                return c2

            lax.fori_loop(0, _CHUNK // 2, row_body, 0, unroll=2)
            writeout(j, b).start()
            nj = j + _NBUF
            if nj < n_chunks:
                writeout(j, b).wait()
                gather_chunk(nj, b)

        for j in range(n_chunks - _NBUF, n_chunks):
            writeout(j, j % _NBUF).wait()

    return k


def kernel(token_ids, embed_weight, scale):
    shape = token_ids.shape
    idx = token_ids.reshape(-1).astype(jnp.int32)
    scale_vec = jnp.broadcast_to(scale.astype(jnp.float32), (16,))
    out = _build(idx.shape[0])(idx, embed_weight, scale_vec)
    return out.reshape(*shape, _D)
